# baseline (speedup 1.0000x reference)
import jax
import jax.numpy as jnp
from jax.experimental import pallas as pl
from jax.experimental.pallas import tpu as pltpu

SUBLANE = 8


def _round_up(x, m):
    return ((x + m - 1) // m) * m


def _mlp_kernel(x_ref, w1_ref, w2_ref, w3_ref, b_ref, o_ref):
    f_p = w1_ref.shape[1]
    out_p = w3_ref.shape[1]
    n_out = o_ref.shape[-1]

    x = x_ref[...].astype(jnp.bfloat16)
    h = jnp.dot(x, w1_ref[...], preferred_element_type=jnp.float32)
    h = jnp.maximum(h + b_ref[0:1, 0:f_p], 0.0).astype(jnp.bfloat16)
    h = jnp.dot(h, w2_ref[...], preferred_element_type=jnp.float32)
    h = jnp.maximum(h + b_ref[1:2, 0:f_p], 0.0).astype(jnp.bfloat16)
    a = jnp.dot(h, w3_ref[...], preferred_element_type=jnp.float32)
    o_ref[...] = (a + b_ref[2:3, 0:out_p])[:, :n_out]


def kernel(state, w1, w2, w3, b, *, block_b=1024):
    if state.ndim == 2:
        state = state[:, None, :]
    B, _, n_in = state.shape
    n_output = 128
    f_p = w1.shape[1]
    out_p = w3.shape[1]

    w1b = w1.astype(jnp.bfloat16)
    w2b = w2.astype(jnp.bfloat16)
    w3b = w3.astype(jnp.bfloat16)

    tb = max(SUBLANE, min(block_b, _round_up(-(-B // 2), SUBLANE)))
    grid = (pl.cdiv(B, tb),)

    flops = 2 * B * (n_in * f_p + f_p * f_p + f_p * out_p)
    bytes_accessed = (
        state.size * state.dtype.itemsize
        + sum(a.size * a.dtype.itemsize for a in (w1b, w2b, w3b, b))
        + B * n_output * 4
    )

    return pl.pallas_call(
        _mlp_kernel,
        out_shape=jax.ShapeDtypeStruct((B, n_output), jnp.float32),
        grid=grid,
        in_specs=[
            pl.BlockSpec((tb, None, n_in), lambda i: (i, 0, 0)),
            pl.BlockSpec(w1b.shape, lambda i: (0, 0)),
            pl.BlockSpec(w2b.shape, lambda i: (0, 0)),
            pl.BlockSpec(w3b.shape, lambda i: (0, 0)),
            pl.BlockSpec(b.shape, lambda i: (0, 0)),
        ],
        out_specs=pl.BlockSpec((tb, n_output), lambda i: (i, 0)),
        compiler_params=pltpu.CompilerParams(
            dimension_semantics=("parallel",)),
        cost_estimate=pl.CostEstimate(
            flops=flops, transcendentals=0, bytes_accessed=bytes_accessed),
    )(state, w1b, w2b, w3b, b)


# speedup vs baseline: 1.6562x; 1.6562x over previous
import jax
import jax.numpy as jnp
from jax.experimental import pallas as pl
from jax.experimental.pallas import tpu as pltpu

SUBLANE = 8


def _round_up(x, m):
    return ((x + m - 1) // m) * m


def _mlp_kernel(x_ref, w1_ref, w2_ref, w3_ref, b_ref, o_ref):
    f_p = w1_ref.shape[1]
    out_p = w3_ref.shape[1]
    n_out = o_ref.shape[-1]

    x = x_ref[...]
    h = jnp.dot(x, w1_ref[...], preferred_element_type=jnp.float32)
    h = jnp.maximum(h + b_ref[0:1, 0:f_p], 0.0)
    h = jnp.dot(h, w2_ref[...], preferred_element_type=jnp.float32)
    h = jnp.maximum(h + b_ref[1:2, 0:f_p], 0.0)
    a = jnp.dot(h, w3_ref[...], preferred_element_type=jnp.float32)
    o_ref[...] = (a + b_ref[2:3, 0:out_p])[:, :n_out]


def kernel(state, w1, w2, w3, b, *, block_b=4096):
    if state.ndim == 2:
        state = state[:, None, :]
    B, _, n_in = state.shape
    n_output = 128
    f_p = w1.shape[1]
    out_p = w3.shape[1]

    tb = max(SUBLANE, min(block_b, _round_up(-(-B // 2), SUBLANE)))
    grid = (pl.cdiv(B, tb),)

    flops = 2 * B * (n_in * f_p + f_p * f_p + f_p * out_p)
    bytes_accessed = (
        state.size * state.dtype.itemsize
        + sum(a.size * a.dtype.itemsize for a in (w1, w2, w3, b))
        + B * n_output * 4
    )

    return pl.pallas_call(
        _mlp_kernel,
        out_shape=jax.ShapeDtypeStruct((B, n_output), jnp.float32),
        grid=grid,
        in_specs=[
            pl.BlockSpec((tb, None, n_in), lambda i: (i, 0, 0)),
            pl.BlockSpec(w1.shape, lambda i: (0, 0)),
            pl.BlockSpec(w2.shape, lambda i: (0, 0)),
            pl.BlockSpec(w3.shape, lambda i: (0, 0)),
            pl.BlockSpec(b.shape, lambda i: (0, 0)),
        ],
        out_specs=pl.BlockSpec((tb, n_output), lambda i: (i, 0)),
        compiler_params=pltpu.CompilerParams(
            dimension_semantics=("parallel",)),
        cost_estimate=pl.CostEstimate(
            flops=flops, transcendentals=0, bytes_accessed=bytes_accessed),
    )(state, w1, w2, w3, b)


# speedup vs baseline: 1.6581x; 1.0012x over previous
import jax
import jax.numpy as jnp
from jax.experimental import pallas as pl
from jax.experimental.pallas import tpu as pltpu

SUBLANE = 8


def _round_up(x, m):
    return ((x + m - 1) // m) * m


def _mlp_kernel(x_ref, w1_ref, w2_ref, w3_ref, b_ref, o_ref):
    f_p = w1_ref.shape[1]
    out_p = w3_ref.shape[1]
    n_out = o_ref.shape[-1]

    x = x_ref[...]
    h = jnp.dot(x, w1_ref[...], preferred_element_type=jnp.float32)
    h = jnp.maximum(h + b_ref[0:1, 0:f_p], 0.0)
    h = jnp.dot(h, w2_ref[...], preferred_element_type=jnp.float32)
    h = jnp.maximum(h + b_ref[1:2, 0:f_p], 0.0)
    a = jnp.dot(h, w3_ref[...], preferred_element_type=jnp.float32)
    o_ref[...] = (a + b_ref[2:3, 0:out_p])[:, :n_out]


def kernel(state, w1, w2, w3, b, *, block_b=8192):
    if state.ndim == 2:
        state = state[:, None, :]
    B, _, n_in = state.shape
    n_output = 128
    f_p = w1.shape[1]
    out_p = w3.shape[1]

    tb = max(SUBLANE, min(block_b, _round_up(-(-B // 2), SUBLANE)))
    grid = (pl.cdiv(B, tb),)

    flops = 2 * B * (n_in * f_p + f_p * f_p + f_p * out_p)
    bytes_accessed = (
        state.size * state.dtype.itemsize
        + sum(a.size * a.dtype.itemsize for a in (w1, w2, w3, b))
        + B * n_output * 4
    )

    return pl.pallas_call(
        _mlp_kernel,
        out_shape=jax.ShapeDtypeStruct((B, n_output), jnp.float32),
        grid=grid,
        in_specs=[
            pl.BlockSpec((tb, None, n_in), lambda i: (i, 0, 0)),
            pl.BlockSpec(w1.shape, lambda i: (0, 0)),
            pl.BlockSpec(w2.shape, lambda i: (0, 0)),
            pl.BlockSpec(w3.shape, lambda i: (0, 0)),
            pl.BlockSpec(b.shape, lambda i: (0, 0)),
        ],
        out_specs=pl.BlockSpec((tb, n_output), lambda i: (i, 0)),
        compiler_params=pltpu.CompilerParams(
            dimension_semantics=("parallel",)),
        cost_estimate=pl.CostEstimate(
            flops=flops, transcendentals=0, bytes_accessed=bytes_accessed),
    )(state, w1, w2, w3, b)
